# baseline (speedup 1.0000x reference)
import functools

import jax
import jax.numpy as jnp
from jax import lax
from jax.experimental import pallas as pl
from jax.experimental.pallas import tpu as pltpu
from jax.experimental.pallas import tpu_sc as plsc

N = 10000
E = 320000
D = 128
K = 50
CUT = 5.0
GAMMA = K / CUT
NUM_TYPES = 101
L = 3
KP = 128

NC = 2
NS = 16
NW = NC * NS
EW = E // NW
EC = 80
NCHUNK = EW // EC
NP = 10240
NR = NP // NS

BE = 2000
BN = 2000

_LOG2 = 0.6931471805599453


def _ssp(x):
    return jnp.maximum(x, 0.0) + jnp.log1p(jnp.exp(-jnp.abs(x))) - _LOG2


def _embed_body(nd_ref, emb_ref, wn_ref, bn_ref, out_ref):
    t = nd_ref[...]
    oh = (t == lax.broadcasted_iota(jnp.int32, (BN, 128), 1)).astype(jnp.float32)
    h0 = jnp.dot(oh, emb_ref[...], preferred_element_type=jnp.float32, precision=lax.Precision.HIGHEST)
    out_ref[...] = (
        jnp.dot(h0, wn_ref[...], preferred_element_type=jnp.float32, precision=lax.Precision.HIGHEST) + bn_ref[...]
    )


def _embed_project(ndata, emb_pad, wn0, bn0):
    return pl.pallas_call(
        _embed_body,
        grid=(N // BN,),
        in_specs=[
            pl.BlockSpec((BN, 1), lambda i: (i, 0)),
            pl.BlockSpec((128, D), lambda i: (0, 0)),
            pl.BlockSpec((D, D), lambda i: (0, 0)),
            pl.BlockSpec((1, D), lambda i: (0, 0)),
        ],
        out_specs=pl.BlockSpec((BN, D), lambda i: (i, 0)),
        out_shape=jax.ShapeDtypeStruct((N, D), jnp.float32),
    )(ndata, emb_pad, wn0, bn0)


def _edge_body(ed_ref, we1_ref, be1_ref, we2_ref, be2_ref, out_ref):
    dist = ed_ref[...]
    centers = lax.broadcasted_iota(jnp.int32, (BE, KP), 1).astype(jnp.float32) * (
        CUT / (K - 1))
    diff = dist - centers
    rbf = jnp.exp(-GAMMA * diff * diff)
    t = _ssp(
        jnp.dot(rbf, we1_ref[...], preferred_element_type=jnp.float32, precision=lax.Precision.HIGHEST)
        + be1_ref[...]
    )
    out_ref[...] = _ssp(
        jnp.dot(t, we2_ref[...], preferred_element_type=jnp.float32, precision=lax.Precision.HIGHEST)
        + be2_ref[...]
    )


def _edge_mlp(edata, we1_pad_l, be1r_l, we2_l, be2r_l):
    return pl.pallas_call(
        _edge_body,
        grid=(E // BE,),
        in_specs=[
            pl.BlockSpec((BE, 1), lambda i: (i, 0)),
            pl.BlockSpec((KP, D), lambda i: (0, 0)),
            pl.BlockSpec((1, D), lambda i: (0, 0)),
            pl.BlockSpec((D, D), lambda i: (0, 0)),
            pl.BlockSpec((1, D), lambda i: (0, 0)),
        ],
        out_specs=pl.BlockSpec((BE, D), lambda i: (i, 0)),
        out_shape=jax.ShapeDtypeStruct((E, D), jnp.float32),
    )(edata, we1_pad_l, be1r_l, we2_l, be2r_l)


def _sc_body(hv_hbm, he_hbm, src_hbm, dst_hbm, out_hbm,
             sidx0, sidx1, didx0, didx1, hvg0, hvg1, hev0, hev1, acc,
             issem, idsem, gsem0, gsem1, hsem0, hsem1, ssem0, ssem1):
    c = lax.axis_index("c")
    s = lax.axis_index("s")
    sidx = (sidx0, sidx1)
    didx = (didx0, didx1)
    hvg = (hvg0, hvg1)
    hev = (hev0, hev1)
    gsem = (gsem0, gsem1)
    hsem = (hsem0, hsem1)
    ssem = (ssem0, ssem1)

    def zrow(r, carry):
        for j in range(D // 16):
            hvg0[r, pl.ds(j * 16, 16)] = jnp.zeros((16,), jnp.float32)
        return carry

    lax.fori_loop(0, EC, zrow, 0)
    for q in range(NR // EC):
        pltpu.sync_copy(hvg0, acc.at[pl.ds(s * NR + q * EC, EC)])
    plsc.subcore_barrier()

    wid = c * NS + s
    base = wid * EW

    def off(k):
        return base + k * EC

    def issue_fetch(k, p):
        pltpu.async_copy(hv_hbm.at[sidx[p]], hvg[p], gsem[p])
        pltpu.async_copy(he_hbm.at[pl.ds(off(k), EC)], hev[p], hsem[p])

    def wait_fetch(k, p):
        pltpu.make_async_copy(hv_hbm.at[sidx[p]], hvg[p], gsem[p]).wait()
        pltpu.make_async_copy(he_hbm.at[pl.ds(off(k), EC)], hev[p], hsem[p]).wait()

    def wait_scatter(p):
        pltpu.make_async_copy(hev[p], acc.at[didx[p]], ssem[p]).wait()

    def multiply(p):
        hv_b, he_b = hvg[p], hev[p]

        @plsc.parallel_loop(0, EC, step=1, unroll=4)
        def mrow(r):
            for j in range(D // 16):
                sl = pl.ds(j * 16, 16)
                he_b[r, sl] = he_b[r, sl] * hv_b[r, sl]

    def chunk_step(k, p, first=False, last=False, skip_src2=False):
        q = 1 - p
        if not first:
            wait_scatter(q)
        if not last:
            pltpu.make_async_copy(src_hbm.at[pl.ds(off(k + 1), EC)],
                                  sidx[q], issem).wait()
            issue_fetch(k + 1, q)
        wait_fetch(k, p)
        if not first:
            pltpu.make_async_copy(dst_hbm.at[pl.ds(off(k), EC)],
                                  didx[p], idsem).wait()
        if not last:
            if not skip_src2:
                pltpu.async_copy(src_hbm.at[pl.ds(off(k + 2), EC)],
                                 sidx[p], issem)
            pltpu.async_copy(dst_hbm.at[pl.ds(off(k + 1), EC)],
                             didx[q], idsem)
        multiply(p)
        pltpu.async_copy(hev[p], acc.at[didx[p]], ssem[p], add=True)

    pltpu.sync_copy(src_hbm.at[pl.ds(off(0), EC)], sidx[0])
    pltpu.sync_copy(dst_hbm.at[pl.ds(off(0), EC)], didx[0])
    issue_fetch(0, 0)
    pltpu.async_copy(src_hbm.at[pl.ds(off(1), EC)], sidx[1], issem)

    chunk_step(0, 0, first=True)

    def pair(t, carry):
        chunk_step(2 * t + 1, 1)
        chunk_step(2 * t + 2, 0)
        return carry

    lax.fori_loop(0, (NCHUNK - 3) // 2, pair, 0)
    chunk_step(NCHUNK - 2, 1, skip_src2=True)
    chunk_step(NCHUNK - 1, 0, last=True)
    wait_scatter(0)

    plsc.subcore_barrier()
    pltpu.sync_copy(acc.at[pl.ds(s * NR, NR)], out_hbm.at[c, pl.ds(s * NR, NR)])


def _make_sc_message_pass():
    mesh = plsc.VectorSubcoreMesh(core_axis_name="c", subcore_axis_name="s")
    return pl.kernel(
        _sc_body,
        out_type=jax.ShapeDtypeStruct((NC, NP, D), jnp.float32),
        mesh=mesh,
        scratch_types=[
            pltpu.VMEM((EC,), jnp.int32),
            pltpu.VMEM((EC,), jnp.int32),
            pltpu.VMEM((EC,), jnp.int32),
            pltpu.VMEM((EC,), jnp.int32),
            pltpu.VMEM((EC, D), jnp.float32),
            pltpu.VMEM((EC, D), jnp.float32),
            pltpu.VMEM((EC, D), jnp.float32),
            pltpu.VMEM((EC, D), jnp.float32),
            pltpu.VMEM_SHARED((NP, D), jnp.float32),
            pltpu.SemaphoreType.DMA,
            pltpu.SemaphoreType.DMA,
            pltpu.SemaphoreType.DMA,
            pltpu.SemaphoreType.DMA,
            pltpu.SemaphoreType.DMA,
            pltpu.SemaphoreType.DMA,
            pltpu.SemaphoreType.DMA,
            pltpu.SemaphoreType.DMA,
        ],
    )


_sc_cache = []


def _sc_message_pass(*args):
    if not _sc_cache:
        _sc_cache.append(_make_sc_message_pass())
    return _sc_cache[0](*args)


def _node_body(p_ref, wo_ref, bo_ref, wp_ref, bp_ref, wn_ref, bn_ref, out_ref):
    agg = p_ref[0] + p_ref[1]
    conv = _ssp(
        jnp.dot(agg, wo_ref[...], preferred_element_type=jnp.float32, precision=lax.Precision.HIGHEST) + bo_ref[...]
    )
    h = jnp.dot(conv, wp_ref[...], preferred_element_type=jnp.float32, precision=lax.Precision.HIGHEST) + bp_ref[...]
    out_ref[...] = (
        jnp.dot(h, wn_ref[...], preferred_element_type=jnp.float32, precision=lax.Precision.HIGHEST) + bn_ref[...]
    )


def _node_update(part, wo, bo_r, wp, bp_r, wn_next, bn_next):
    return pl.pallas_call(
        _node_body,
        grid=(N // BN,),
        in_specs=[
            pl.BlockSpec((NC, BN, D), lambda i: (0, i, 0)),
            pl.BlockSpec((D, D), lambda i: (0, 0)),
            pl.BlockSpec((1, D), lambda i: (0, 0)),
            pl.BlockSpec((D, D), lambda i: (0, 0)),
            pl.BlockSpec((1, D), lambda i: (0, 0)),
            pl.BlockSpec((D, D), lambda i: (0, 0)),
            pl.BlockSpec((1, D), lambda i: (0, 0)),
        ],
        out_specs=pl.BlockSpec((BN, D), lambda i: (i, 0)),
        out_shape=jax.ShapeDtypeStruct((N, D), jnp.float32),
    )(part, wo, bo_r, wp, bp_r, wn_next, bn_next)


def _final_body(p_ref, wo_ref, bo_ref, wp_ref, bp_ref, w1_ref, b1_ref,
                w2_ref, b2_ref, m_ref, out_ref):
    agg = p_ref[0] + p_ref[1]
    conv = _ssp(
        jnp.dot(agg, wo_ref[...], preferred_element_type=jnp.float32, precision=lax.Precision.HIGHEST) + bo_ref[...]
    )
    h = jnp.dot(conv, wp_ref[...], preferred_element_type=jnp.float32, precision=lax.Precision.HIGHEST) + bp_ref[...]
    e = jnp.maximum(
        jnp.dot(h, w1_ref[...], preferred_element_type=jnp.float32, precision=lax.Precision.HIGHEST) + b1_ref[...],
        0.0,
    )
    e2 = jnp.sum(e * w2_ref[...], axis=1, keepdims=True) + b2_ref[...]
    e2 = e2 * m_ref[...]
    blk = jnp.sum(e2, axis=0, keepdims=True)

    @pl.when(pl.program_id(0) == 0)
    def _init():
        out_ref[...] = jnp.zeros_like(out_ref)

    out_ref[...] += blk


def _final_readout(part, wo, bo_r, wp, bp_r, w1_pad, b1_pad, w2_row, b2_r, mask):
    return pl.pallas_call(
        _final_body,
        grid=(N // BN,),
        in_specs=[
            pl.BlockSpec((NC, BN, D), lambda i: (0, i, 0)),
            pl.BlockSpec((D, D), lambda i: (0, 0)),
            pl.BlockSpec((1, D), lambda i: (0, 0)),
            pl.BlockSpec((D, D), lambda i: (0, 0)),
            pl.BlockSpec((1, D), lambda i: (0, 0)),
            pl.BlockSpec((D, 128), lambda i: (0, 0)),
            pl.BlockSpec((1, 128), lambda i: (0, 0)),
            pl.BlockSpec((1, 128), lambda i: (0, 0)),
            pl.BlockSpec((1, 1), lambda i: (0, 0)),
            pl.BlockSpec((BN, 1), lambda i: (i, 0)),
        ],
        out_specs=pl.BlockSpec((1, 1), lambda i: (0, 0)),
        out_shape=jax.ShapeDtypeStruct((1, 1), jnp.float32),
    )(part, wo, bo_r, wp, bp_r, w1_pad, b1_pad, w2_row, b2_r, mask)


def kernel(batch_g, ndata, edata, is_contributing, embed, Wn, bn, We1, be1,
           We2, be2, Wo, bo, Wp, bp, W1o, b1o, W2o, b2o):
    src = batch_g[0]
    dst = batch_g[1]

    emb_pad = jnp.zeros((128, D), jnp.float32).at[:NUM_TYPES].set(embed)
    we1_pad = jnp.zeros((L, KP, D), jnp.float32).at[:, :K].set(We1)
    be1r = be1.reshape(L, 1, D)
    be2r = be2.reshape(L, 1, D)
    bnr = bn.reshape(L, 1, D)
    bor = bo.reshape(L, 1, D)
    bpr = bp.reshape(L, 1, D)
    w1_pad = jnp.zeros((D, 128), jnp.float32).at[:, : D // 2].set(W1o)
    b1_pad = jnp.zeros((1, 128), jnp.float32).at[:, : D // 2].set(b1o)
    w2_row = jnp.zeros((1, 128), jnp.float32).at[0, : D // 2].set(W2o[:, 0])
    b2_r = b2o.reshape(1, 1)

    hv = _embed_project(ndata, emb_pad, Wn[0], bnr[0])

    he_i = _edge_mlp(edata, we1_pad[0], be1r[0], We2[0], be2r[0])
    for i in range(L):
        part = _sc_message_pass(hv, he_i, src, dst)
        if i < L - 1:
            he_i = _edge_mlp(edata, we1_pad[i + 1], be1r[i + 1],
                             We2[i + 1], be2r[i + 1])
            hv = _node_update(part, Wo[i], bor[i], Wp[i], bpr[i],
                              Wn[i + 1], bnr[i + 1])
        else:
            out = _final_readout(part, Wo[i], bor[i], Wp[i], bpr[i],
                                 w1_pad, b1_pad, w2_row, b2_r, is_contributing)
    return out.reshape((1,))


# speedup vs baseline: 2.0879x; 2.0879x over previous
import functools

import jax
import jax.numpy as jnp
from jax import lax
from jax.experimental import pallas as pl
from jax.experimental.pallas import tpu as pltpu
from jax.experimental.pallas import tpu_sc as plsc

N = 10000
E = 320000
D = 128
K = 50
CUT = 5.0
GAMMA = K / CUT
NUM_TYPES = 101
L = 3
KP = 128

NC = 2
NS = 16
NW = NC * NS
EW = E // NW
EC = 80
NCHUNK = EW // EC
NP = 10240
NR = NP // NS

BE = 2000
BN = 2000

_LOG2 = 0.6931471805599453


def _ssp(x):
    return jnp.maximum(x, 0.0) + jnp.log1p(jnp.exp(-jnp.abs(x))) - _LOG2


def _embed_body(nd_ref, emb_ref, wn_ref, bn_ref, out_ref):
    t = nd_ref[...]
    oh = (t == lax.broadcasted_iota(jnp.int32, (BN, 128), 1)).astype(jnp.float32)
    h0 = jnp.dot(oh, emb_ref[...], preferred_element_type=jnp.float32)
    out_ref[...] = (
        jnp.dot(h0, wn_ref[...], preferred_element_type=jnp.float32) + bn_ref[...]
    )


def _embed_project(ndata, emb_pad, wn0, bn0):
    return pl.pallas_call(
        _embed_body,
        grid=(N // BN,),
        in_specs=[
            pl.BlockSpec((BN, 1), lambda i: (i, 0)),
            pl.BlockSpec((128, D), lambda i: (0, 0)),
            pl.BlockSpec((D, D), lambda i: (0, 0)),
            pl.BlockSpec((1, D), lambda i: (0, 0)),
        ],
        out_specs=pl.BlockSpec((BN, D), lambda i: (i, 0)),
        out_shape=jax.ShapeDtypeStruct((N, D), jnp.float32),
    )(ndata, emb_pad, wn0, bn0)


def _edge_body(ed_ref, cen_ref, we1_ref, be1_ref, we2_ref, be2_ref, out_ref):
    dist = ed_ref[...]
    diff = dist - cen_ref[...]
    rbf = jnp.exp((-GAMMA) * (diff * diff))
    t = _ssp(
        jnp.dot(rbf, we1_ref[...], preferred_element_type=jnp.float32)
        + be1_ref[...]
    )
    out_ref[...] = _ssp(
        jnp.dot(t, we2_ref[...], preferred_element_type=jnp.float32)
        + be2_ref[...]
    )


def _edge_mlp(edata, cen_row, we1_pad_l, be1r_l, we2_l, be2r_l):
    return pl.pallas_call(
        _edge_body,
        grid=(E // BE,),
        in_specs=[
            pl.BlockSpec((BE, 1), lambda i: (i, 0)),
            pl.BlockSpec((1, KP), lambda i: (0, 0)),
            pl.BlockSpec((KP, D), lambda i: (0, 0)),
            pl.BlockSpec((1, D), lambda i: (0, 0)),
            pl.BlockSpec((D, D), lambda i: (0, 0)),
            pl.BlockSpec((1, D), lambda i: (0, 0)),
        ],
        out_specs=pl.BlockSpec((BE, D), lambda i: (i, 0)),
        out_shape=jax.ShapeDtypeStruct((E, D), jnp.float32),
    )(edata, cen_row, we1_pad_l, be1r_l, we2_l, be2r_l)


def _sc_body(hv_hbm, he_hbm, src_hbm, dst_hbm, out_hbm,
             sidx0, sidx1, didx0, didx1, hvg0, hvg1, hev0, hev1, acc,
             issem, idsem, gsem0, gsem1, hsem0, hsem1, ssem0, ssem1):
    c = lax.axis_index("c")
    s = lax.axis_index("s")
    sidx = (sidx0, sidx1)
    didx = (didx0, didx1)
    hvg = (hvg0, hvg1)
    hev = (hev0, hev1)
    gsem = (gsem0, gsem1)
    hsem = (hsem0, hsem1)
    ssem = (ssem0, ssem1)

    def zrow(r, carry):
        for j in range(D // 16):
            hvg0[r, pl.ds(j * 16, 16)] = jnp.zeros((16,), jnp.float32)
        return carry

    lax.fori_loop(0, EC, zrow, 0)
    for q in range(NR // EC):
        pltpu.sync_copy(hvg0, acc.at[pl.ds(s * NR + q * EC, EC)])
    plsc.subcore_barrier()

    wid = c * NS + s
    base = wid * EW

    def off(k):
        return base + k * EC

    def issue_fetch(k, p):
        pltpu.async_copy(hv_hbm.at[sidx[p]], hvg[p], gsem[p])
        pltpu.async_copy(he_hbm.at[pl.ds(off(k), EC)], hev[p], hsem[p])

    def wait_fetch(k, p):
        pltpu.make_async_copy(hv_hbm.at[sidx[p]], hvg[p], gsem[p]).wait()
        pltpu.make_async_copy(he_hbm.at[pl.ds(off(k), EC)], hev[p], hsem[p]).wait()

    def wait_scatter(p):
        pltpu.make_async_copy(hev[p], acc.at[didx[p]], ssem[p]).wait()

    def multiply(p):
        hv_b, he_b = hvg[p], hev[p]

        @plsc.parallel_loop(0, EC, step=1, unroll=4)
        def mrow(r):
            for j in range(D // 16):
                sl = pl.ds(j * 16, 16)
                he_b[r, sl] = he_b[r, sl] * hv_b[r, sl]

    def chunk_step(k, p, first=False, last=False, skip_src2=False):
        q = 1 - p
        if not first:
            wait_scatter(q)
        if not last:
            pltpu.make_async_copy(src_hbm.at[pl.ds(off(k + 1), EC)],
                                  sidx[q], issem).wait()
            issue_fetch(k + 1, q)
        wait_fetch(k, p)
        if not first:
            pltpu.make_async_copy(dst_hbm.at[pl.ds(off(k), EC)],
                                  didx[p], idsem).wait()
        if not last:
            if not skip_src2:
                pltpu.async_copy(src_hbm.at[pl.ds(off(k + 2), EC)],
                                 sidx[p], issem)
            pltpu.async_copy(dst_hbm.at[pl.ds(off(k + 1), EC)],
                             didx[q], idsem)
        multiply(p)
        pltpu.async_copy(hev[p], acc.at[didx[p]], ssem[p], add=True)

    pltpu.sync_copy(src_hbm.at[pl.ds(off(0), EC)], sidx[0])
    pltpu.sync_copy(dst_hbm.at[pl.ds(off(0), EC)], didx[0])
    issue_fetch(0, 0)
    pltpu.async_copy(src_hbm.at[pl.ds(off(1), EC)], sidx[1], issem)

    chunk_step(0, 0, first=True)

    def pair(t, carry):
        chunk_step(2 * t + 1, 1)
        chunk_step(2 * t + 2, 0)
        return carry

    lax.fori_loop(0, (NCHUNK - 3) // 2, pair, 0)
    chunk_step(NCHUNK - 2, 1, skip_src2=True)
    chunk_step(NCHUNK - 1, 0, last=True)
    wait_scatter(0)

    plsc.subcore_barrier()
    pltpu.sync_copy(acc.at[pl.ds(s * NR, NR)], out_hbm.at[c, pl.ds(s * NR, NR)])


def _make_sc_message_pass():
    mesh = plsc.VectorSubcoreMesh(core_axis_name="c", subcore_axis_name="s")
    return pl.kernel(
        _sc_body,
        out_type=jax.ShapeDtypeStruct((NC, NP, D), jnp.float32),
        mesh=mesh,
        scratch_types=[
            pltpu.VMEM((EC,), jnp.int32),
            pltpu.VMEM((EC,), jnp.int32),
            pltpu.VMEM((EC,), jnp.int32),
            pltpu.VMEM((EC,), jnp.int32),
            pltpu.VMEM((EC, D), jnp.float32),
            pltpu.VMEM((EC, D), jnp.float32),
            pltpu.VMEM((EC, D), jnp.float32),
            pltpu.VMEM((EC, D), jnp.float32),
            pltpu.VMEM_SHARED((NP, D), jnp.float32),
            pltpu.SemaphoreType.DMA,
            pltpu.SemaphoreType.DMA,
            pltpu.SemaphoreType.DMA,
            pltpu.SemaphoreType.DMA,
            pltpu.SemaphoreType.DMA,
            pltpu.SemaphoreType.DMA,
            pltpu.SemaphoreType.DMA,
            pltpu.SemaphoreType.DMA,
        ],
    )


_sc_cache = []


def _sc_message_pass(*args):
    if not _sc_cache:
        _sc_cache.append(_make_sc_message_pass())
    return _sc_cache[0](*args)


def _node_body(p_ref, wo_ref, bo_ref, wp_ref, bp_ref, wn_ref, bn_ref, out_ref):
    agg = p_ref[0] + p_ref[1]
    conv = _ssp(
        jnp.dot(agg, wo_ref[...], preferred_element_type=jnp.float32) + bo_ref[...]
    )
    h = jnp.dot(conv, wp_ref[...], preferred_element_type=jnp.float32) + bp_ref[...]
    out_ref[...] = (
        jnp.dot(h, wn_ref[...], preferred_element_type=jnp.float32) + bn_ref[...]
    )


def _node_update(part, wo, bo_r, wp, bp_r, wn_next, bn_next):
    return pl.pallas_call(
        _node_body,
        grid=(N // BN,),
        in_specs=[
            pl.BlockSpec((NC, BN, D), lambda i: (0, i, 0)),
            pl.BlockSpec((D, D), lambda i: (0, 0)),
            pl.BlockSpec((1, D), lambda i: (0, 0)),
            pl.BlockSpec((D, D), lambda i: (0, 0)),
            pl.BlockSpec((1, D), lambda i: (0, 0)),
            pl.BlockSpec((D, D), lambda i: (0, 0)),
            pl.BlockSpec((1, D), lambda i: (0, 0)),
        ],
        out_specs=pl.BlockSpec((BN, D), lambda i: (i, 0)),
        out_shape=jax.ShapeDtypeStruct((N, D), jnp.float32),
    )(part, wo, bo_r, wp, bp_r, wn_next, bn_next)


def _final_body(p_ref, wo_ref, bo_ref, wp_ref, bp_ref, w1_ref, b1_ref,
                w2_ref, b2_ref, m_ref, out_ref):
    agg = p_ref[0] + p_ref[1]
    conv = _ssp(
        jnp.dot(agg, wo_ref[...], preferred_element_type=jnp.float32) + bo_ref[...]
    )
    h = jnp.dot(conv, wp_ref[...], preferred_element_type=jnp.float32) + bp_ref[...]
    e = jnp.maximum(
        jnp.dot(h, w1_ref[...], preferred_element_type=jnp.float32) + b1_ref[...],
        0.0,
    )
    e2 = jnp.sum(e * w2_ref[...], axis=1, keepdims=True) + b2_ref[...]
    e2 = e2 * m_ref[...]
    blk = jnp.sum(e2, axis=0, keepdims=True)

    @pl.when(pl.program_id(0) == 0)
    def _init():
        out_ref[...] = jnp.zeros_like(out_ref)

    out_ref[...] += blk


def _final_readout(part, wo, bo_r, wp, bp_r, w1_pad, b1_pad, w2_row, b2_r, mask):
    return pl.pallas_call(
        _final_body,
        grid=(N // BN,),
        in_specs=[
            pl.BlockSpec((NC, BN, D), lambda i: (0, i, 0)),
            pl.BlockSpec((D, D), lambda i: (0, 0)),
            pl.BlockSpec((1, D), lambda i: (0, 0)),
            pl.BlockSpec((D, D), lambda i: (0, 0)),
            pl.BlockSpec((1, D), lambda i: (0, 0)),
            pl.BlockSpec((D, 128), lambda i: (0, 0)),
            pl.BlockSpec((1, 128), lambda i: (0, 0)),
            pl.BlockSpec((1, 128), lambda i: (0, 0)),
            pl.BlockSpec((1, 1), lambda i: (0, 0)),
            pl.BlockSpec((BN, 1), lambda i: (i, 0)),
        ],
        out_specs=pl.BlockSpec((1, 1), lambda i: (0, 0)),
        out_shape=jax.ShapeDtypeStruct((1, 1), jnp.float32),
    )(part, wo, bo_r, wp, bp_r, w1_pad, b1_pad, w2_row, b2_r, mask)


def kernel(batch_g, ndata, edata, is_contributing, embed, Wn, bn, We1, be1,
           We2, be2, Wo, bo, Wp, bp, W1o, b1o, W2o, b2o):
    src = batch_g[0]
    dst = batch_g[1]

    emb_pad = jnp.zeros((128, D), jnp.float32).at[:NUM_TYPES].set(embed)
    centers = jnp.linspace(0.0, CUT, K).astype(jnp.float32)
    cen_row = jnp.full((1, KP), CUT, jnp.float32).at[0, :K].set(centers)
    we1_pad = jnp.zeros((L, KP, D), jnp.float32).at[:, :K].set(We1)
    be1r = be1.reshape(L, 1, D)
    be2r = be2.reshape(L, 1, D)
    bnr = bn.reshape(L, 1, D)
    bor = bo.reshape(L, 1, D)
    bpr = bp.reshape(L, 1, D)
    w1_pad = jnp.zeros((D, 128), jnp.float32).at[:, : D // 2].set(W1o)
    b1_pad = jnp.zeros((1, 128), jnp.float32).at[:, : D // 2].set(b1o)
    w2_row = jnp.zeros((1, 128), jnp.float32).at[0, : D // 2].set(W2o[:, 0])
    b2_r = b2o.reshape(1, 1)

    hv = _embed_project(ndata, emb_pad, Wn[0], bnr[0])

    he_i = _edge_mlp(edata, cen_row, we1_pad[0], be1r[0], We2[0], be2r[0])
    for i in range(L):
        part = _sc_message_pass(hv, he_i, src, dst)
        if i < L - 1:
            he_i = _edge_mlp(edata, cen_row, we1_pad[i + 1], be1r[i + 1],
                             We2[i + 1], be2r[i + 1])
            hv = _node_update(part, Wo[i], bor[i], Wp[i], bpr[i],
                              Wn[i + 1], bnr[i + 1])
        else:
            out = _final_readout(part, Wo[i], bor[i], Wp[i], bpr[i],
                                 w1_pad, b1_pad, w2_row, b2_r, is_contributing)
    return out.reshape((1,))


# speedup vs baseline: 2.3113x; 1.1070x over previous
import jax
import jax.numpy as jnp
from jax import lax
from jax.experimental import pallas as pl
from jax.experimental.pallas import tpu as pltpu
from jax.experimental.pallas import tpu_sc as plsc

N = 10000
E = 320000
D = 128
K = 50
CUT = 5.0
GAMMA = K / CUT
NUM_TYPES = 101
L = 3
KP = 128

NC = 2
NS = 16
NW = NC * NS
EW = E // NW
EC = 80
NCHUNK = EW // EC
NP = 10240
NR = NP // NS

BE = 4000
BN = 2000

_LOG2 = 0.6931471805599453


def _ssp(x):
    return jnp.maximum(x, 0.0) + jnp.log1p(jnp.exp(-jnp.abs(x))) - _LOG2


def _embed_body(nd_ref, emb_ref, wn_ref, bn_ref, out_ref):
    t = nd_ref[...]
    oh = (t == lax.broadcasted_iota(jnp.int32, (BN, 128), 1)).astype(jnp.float32)
    h0 = jnp.dot(oh, emb_ref[...], preferred_element_type=jnp.float32)
    out_ref[...] = (
        jnp.dot(h0, wn_ref[...], preferred_element_type=jnp.float32) + bn_ref[...]
    )


def _embed_project(ndata, emb_pad, wn0, bn0):
    return pl.pallas_call(
        _embed_body,
        grid=(N // BN,),
        in_specs=[
            pl.BlockSpec((BN, 1), lambda i: (i, 0)),
            pl.BlockSpec((128, D), lambda i: (0, 0)),
            pl.BlockSpec((D, D), lambda i: (0, 0)),
            pl.BlockSpec((1, D), lambda i: (0, 0)),
        ],
        out_specs=pl.BlockSpec((BN, D), lambda i: (i, 0)),
        out_shape=jax.ShapeDtypeStruct((N, D), jnp.float32),
    )(ndata, emb_pad, wn0, bn0)


def _edge_body(ed_ref, cen_ref, we1_ref, be1_ref, we2_ref, be2_ref, out_ref):
    dist = ed_ref[...]
    diff = dist - cen_ref[...]
    rbf = jnp.exp((-GAMMA) * (diff * diff))
    t = _ssp(
        jnp.dot(rbf, we1_ref[...], preferred_element_type=jnp.float32)
        + be1_ref[...]
    )
    out_ref[...] = _ssp(
        jnp.dot(t, we2_ref[...], preferred_element_type=jnp.float32)
        + be2_ref[...]
    )


def _edge_mlp(edata, cen_row, we1_pad_l, be1r_l, we2_l, be2r_l):
    return pl.pallas_call(
        _edge_body,
        grid=(E // BE,),
        in_specs=[
            pl.BlockSpec((BE, 1), lambda i: (i, 0)),
            pl.BlockSpec((1, KP), lambda i: (0, 0)),
            pl.BlockSpec((KP, D), lambda i: (0, 0)),
            pl.BlockSpec((1, D), lambda i: (0, 0)),
            pl.BlockSpec((D, D), lambda i: (0, 0)),
            pl.BlockSpec((1, D), lambda i: (0, 0)),
        ],
        out_specs=pl.BlockSpec((BE, D), lambda i: (i, 0)),
        out_shape=jax.ShapeDtypeStruct((E, D), jnp.float32),
    )(edata, cen_row, we1_pad_l, be1r_l, we2_l, be2r_l)


def _sc_body(hv_hbm, he_hbm, src_hbm, dst_hbm, out_hbm,
             sidx0, sidx1, didx0, didx1, hvg0, hvg1, hev0, hev1, acc,
             issem, idsem, gsem0, gsem1, hsem0, hsem1, ssem0, ssem1):
    c = lax.axis_index("c")
    s = lax.axis_index("s")
    sidx = (sidx0, sidx1)
    didx = (didx0, didx1)
    hvg = (hvg0, hvg1)
    hev = (hev0, hev1)
    gsem = (gsem0, gsem1)
    hsem = (hsem0, hsem1)
    ssem = (ssem0, ssem1)

    def zrow(r, carry):
        for j in range(D // 16):
            hvg0[r, pl.ds(j * 16, 16)] = jnp.zeros((16,), jnp.float32)
        return carry

    lax.fori_loop(0, EC, zrow, 0)
    for q in range(NR // EC):
        pltpu.sync_copy(hvg0, acc.at[pl.ds(s * NR + q * EC, EC)])
    plsc.subcore_barrier()

    wid = c * NS + s
    base = wid * EW

    def off(k):
        return base + k * EC

    def issue_fetch(k, p):
        pltpu.async_copy(hv_hbm.at[sidx[p]], hvg[p], gsem[p])
        pltpu.async_copy(he_hbm.at[pl.ds(off(k), EC)], hev[p], hsem[p])

    def wait_fetch(k, p):
        pltpu.make_async_copy(hv_hbm.at[sidx[p]], hvg[p], gsem[p]).wait()
        pltpu.make_async_copy(he_hbm.at[pl.ds(off(k), EC)], hev[p], hsem[p]).wait()

    def wait_scatter(p):
        pltpu.make_async_copy(hev[p], acc.at[didx[p]], ssem[p]).wait()

    def multiply(p):
        hv_b, he_b = hvg[p], hev[p]

        @plsc.parallel_loop(0, EC, step=1, unroll=4)
        def mrow(r):
            for j in range(D // 16):
                sl = pl.ds(j * 16, 16)
                he_b[r, sl] = he_b[r, sl] * hv_b[r, sl]

    def chunk_step(k, p, first=False, last=False, skip_src2=False):
        q = 1 - p
        if not first:
            wait_scatter(q)
        if not last:
            pltpu.make_async_copy(src_hbm.at[pl.ds(off(k + 1), EC)],
                                  sidx[q], issem).wait()
            issue_fetch(k + 1, q)
        wait_fetch(k, p)
        if not first:
            pltpu.make_async_copy(dst_hbm.at[pl.ds(off(k), EC)],
                                  didx[p], idsem).wait()
        if not last:
            if not skip_src2:
                pltpu.async_copy(src_hbm.at[pl.ds(off(k + 2), EC)],
                                 sidx[p], issem)
            pltpu.async_copy(dst_hbm.at[pl.ds(off(k + 1), EC)],
                             didx[q], idsem)
        multiply(p)
        pltpu.async_copy(hev[p], acc.at[didx[p]], ssem[p], add=True)

    pltpu.sync_copy(src_hbm.at[pl.ds(off(0), EC)], sidx[0])
    pltpu.sync_copy(dst_hbm.at[pl.ds(off(0), EC)], didx[0])
    issue_fetch(0, 0)
    pltpu.async_copy(src_hbm.at[pl.ds(off(1), EC)], sidx[1], issem)

    chunk_step(0, 0, first=True)

    def pair(t, carry):
        chunk_step(2 * t + 1, 1)
        chunk_step(2 * t + 2, 0)
        return carry

    lax.fori_loop(0, (NCHUNK - 3) // 2, pair, 0)
    chunk_step(NCHUNK - 2, 1, skip_src2=True)
    chunk_step(NCHUNK - 1, 0, last=True)
    wait_scatter(0)

    plsc.subcore_barrier()
    pltpu.sync_copy(acc.at[pl.ds(s * NR, NR)], out_hbm.at[c, pl.ds(s * NR, NR)])


def _make_sc_message_pass():
    mesh = plsc.VectorSubcoreMesh(core_axis_name="c", subcore_axis_name="s")
    return pl.kernel(
        _sc_body,
        out_type=jax.ShapeDtypeStruct((NC, NP, D), jnp.float32),
        mesh=mesh,
        scratch_types=[
            pltpu.VMEM((EC,), jnp.int32),
            pltpu.VMEM((EC,), jnp.int32),
            pltpu.VMEM((EC,), jnp.int32),
            pltpu.VMEM((EC,), jnp.int32),
            pltpu.VMEM((EC, D), jnp.float32),
            pltpu.VMEM((EC, D), jnp.float32),
            pltpu.VMEM((EC, D), jnp.float32),
            pltpu.VMEM((EC, D), jnp.float32),
            pltpu.VMEM_SHARED((NP, D), jnp.float32),
            pltpu.SemaphoreType.DMA,
            pltpu.SemaphoreType.DMA,
            pltpu.SemaphoreType.DMA,
            pltpu.SemaphoreType.DMA,
            pltpu.SemaphoreType.DMA,
            pltpu.SemaphoreType.DMA,
            pltpu.SemaphoreType.DMA,
            pltpu.SemaphoreType.DMA,
        ],
    )


_sc_cache = []


def _sc_message_pass(*args):
    if not _sc_cache:
        _sc_cache.append(_make_sc_message_pass())
    return _sc_cache[0](*args)


def _node_body(p_ref, wo_ref, bo_ref, wp_ref, bp_ref, wn_ref, bn_ref, out_ref):
    agg = p_ref[0] + p_ref[1]
    conv = _ssp(
        jnp.dot(agg, wo_ref[...], preferred_element_type=jnp.float32) + bo_ref[...]
    )
    h = jnp.dot(conv, wp_ref[...], preferred_element_type=jnp.float32) + bp_ref[...]
    out_ref[...] = (
        jnp.dot(h, wn_ref[...], preferred_element_type=jnp.float32) + bn_ref[...]
    )


def _node_update(part, wo, bo_r, wp, bp_r, wn_next, bn_next):
    return pl.pallas_call(
        _node_body,
        grid=(N // BN,),
        in_specs=[
            pl.BlockSpec((NC, BN, D), lambda i: (0, i, 0)),
            pl.BlockSpec((D, D), lambda i: (0, 0)),
            pl.BlockSpec((1, D), lambda i: (0, 0)),
            pl.BlockSpec((D, D), lambda i: (0, 0)),
            pl.BlockSpec((1, D), lambda i: (0, 0)),
            pl.BlockSpec((D, D), lambda i: (0, 0)),
            pl.BlockSpec((1, D), lambda i: (0, 0)),
        ],
        out_specs=pl.BlockSpec((BN, D), lambda i: (i, 0)),
        out_shape=jax.ShapeDtypeStruct((N, D), jnp.float32),
    )(part, wo, bo_r, wp, bp_r, wn_next, bn_next)


def _final_body(p_ref, wo_ref, bo_ref, wp_ref, bp_ref, w1_ref, b1_ref,
                w2_ref, b2_ref, m_ref, out_ref):
    agg = p_ref[0] + p_ref[1]
    conv = _ssp(
        jnp.dot(agg, wo_ref[...], preferred_element_type=jnp.float32) + bo_ref[...]
    )
    h = jnp.dot(conv, wp_ref[...], preferred_element_type=jnp.float32) + bp_ref[...]
    e = jnp.maximum(
        jnp.dot(h, w1_ref[...], preferred_element_type=jnp.float32) + b1_ref[...],
        0.0,
    )
    e2 = jnp.sum(e * w2_ref[...], axis=1, keepdims=True) + b2_ref[...]
    e2 = e2 * m_ref[...]
    blk = jnp.sum(e2, axis=0, keepdims=True)

    @pl.when(pl.program_id(0) == 0)
    def _init():
        out_ref[...] = jnp.zeros_like(out_ref)

    out_ref[...] += blk


def _final_readout(part, wo, bo_r, wp, bp_r, w1_pad, b1_pad, w2_row, b2_r, mask):
    return pl.pallas_call(
        _final_body,
        grid=(N // BN,),
        in_specs=[
            pl.BlockSpec((NC, BN, D), lambda i: (0, i, 0)),
            pl.BlockSpec((D, D), lambda i: (0, 0)),
            pl.BlockSpec((1, D), lambda i: (0, 0)),
            pl.BlockSpec((D, D), lambda i: (0, 0)),
            pl.BlockSpec((1, D), lambda i: (0, 0)),
            pl.BlockSpec((D, 128), lambda i: (0, 0)),
            pl.BlockSpec((1, 128), lambda i: (0, 0)),
            pl.BlockSpec((1, 128), lambda i: (0, 0)),
            pl.BlockSpec((1, 1), lambda i: (0, 0)),
            pl.BlockSpec((BN, 1), lambda i: (i, 0)),
        ],
        out_specs=pl.BlockSpec((1, 1), lambda i: (0, 0)),
        out_shape=jax.ShapeDtypeStruct((1, 1), jnp.float32),
    )(part, wo, bo_r, wp, bp_r, w1_pad, b1_pad, w2_row, b2_r, mask)


def kernel(batch_g, ndata, edata, is_contributing, embed, Wn, bn, We1, be1,
           We2, be2, Wo, bo, Wp, bp, W1o, b1o, W2o, b2o):
    src = batch_g[0]
    dst = batch_g[1]

    emb_pad = jnp.zeros((128, D), jnp.float32).at[:NUM_TYPES].set(embed)
    centers = jnp.linspace(0.0, CUT, K).astype(jnp.float32)
    cen_row = jnp.full((1, KP), CUT, jnp.float32).at[0, :K].set(centers)
    we1_pad = jnp.zeros((L, KP, D), jnp.float32).at[:, :K].set(We1)
    be1r = be1.reshape(L, 1, D)
    be2r = be2.reshape(L, 1, D)
    bnr = bn.reshape(L, 1, D)
    bor = bo.reshape(L, 1, D)
    bpr = bp.reshape(L, 1, D)
    w1_pad = jnp.zeros((D, 128), jnp.float32).at[:, : D // 2].set(W1o)
    b1_pad = jnp.zeros((1, 128), jnp.float32).at[:, : D // 2].set(b1o)
    w2_row = jnp.zeros((1, 128), jnp.float32).at[0, : D // 2].set(W2o[:, 0])
    b2_r = b2o.reshape(1, 1)

    hv = _embed_project(ndata, emb_pad, Wn[0], bnr[0])

    he_i = _edge_mlp(edata, cen_row, we1_pad[0], be1r[0], We2[0], be2r[0])
    for i in range(L):
        part = _sc_message_pass(hv, he_i, src, dst)
        if i < L - 1:
            he_i = _edge_mlp(edata, cen_row, we1_pad[i + 1], be1r[i + 1],
                             We2[i + 1], be2r[i + 1])
            hv = _node_update(part, Wo[i], bor[i], Wp[i], bpr[i],
                              Wn[i + 1], bnr[i + 1])
        else:
            out = _final_readout(part, Wo[i], bor[i], Wp[i], bpr[i],
                                 w1_pad, b1_pad, w2_row, b2_r, is_contributing)
    return out.reshape((1,))
